# baseline (speedup 1.0000x reference)
import functools

import jax
import jax.numpy as jnp
from jax.experimental import pallas as pl
from jax.experimental.pallas import tpu as pltpu

N_TOK = 16384
D = 2048
E = 64
K = 8
BT = 512


def _router_block(x_ref, w_ref, idx_ref, gates_ref, logits_ref):
    x = x_ref[...]
    w = w_ref[...]
    logits = jax.lax.dot_general(
        x, w, (((1,), (1,)), ((), ())),
        preferred_element_type=jnp.float32,
        precision=jax.lax.Precision.DEFAULT,
    )
    logits_ref[...] = logits

    iota = jax.lax.broadcasted_iota(jnp.int32, (BT, E), 1)
    work = logits
    vals = []
    idxs = []
    for _ in range(K):
        m = jnp.max(work, axis=1, keepdims=True)
        hit = work >= m
        idx = jnp.min(jnp.where(hit, iota, E), axis=1, keepdims=True)
        vals.append(m)
        idxs.append(idx)
        work = jnp.where(iota == idx, -jnp.inf, work)
    v = jnp.concatenate(vals, axis=1)
    gates = jnp.exp(v - v[:, :1])
    gates = gates / jnp.sum(gates, axis=1, keepdims=True)
    idx_ref[...] = jnp.concatenate(idxs, axis=1)
    gates_ref[...] = gates


@functools.partial(jax.jit, static_argnames=())
def kernel(hidden_states, gate_weight):
    grid = (N_TOK // BT,)
    idx, gates, logits = pl.pallas_call(
        _router_block,
        grid=grid,
        in_specs=[
            pl.BlockSpec((BT, D), lambda i: (i, 0)),
            pl.BlockSpec((E, D), lambda i: (0, 0)),
        ],
        out_specs=[
            pl.BlockSpec((BT, K), lambda i: (i, 0)),
            pl.BlockSpec((BT, K), lambda i: (i, 0)),
            pl.BlockSpec((BT, E), lambda i: (i, 0)),
        ],
        out_shape=[
            jax.ShapeDtypeStruct((N_TOK, K), jnp.int32),
            jax.ShapeDtypeStruct((N_TOK, K), jnp.float32),
            jax.ShapeDtypeStruct((N_TOK, E), jnp.float32),
        ],
    )(hidden_states, gate_weight)
    return idx, gates, logits


# speedup vs baseline: 1.3689x; 1.3689x over previous
import functools

import jax
import jax.numpy as jnp
from jax.experimental import pallas as pl
from jax.experimental.pallas import tpu as pltpu

N_TOK = 16384
D = 2048
E = 64
K = 8
BT = 1024


def _router_block(x_ref, w_ref, idx_ref, gates_ref, logits_ref):
    x = x_ref[...]
    w = w_ref[...]
    logits = jax.lax.dot_general(
        x, w, (((1,), (1,)), ((), ())),
        preferred_element_type=jnp.float32,
        precision=jax.lax.Precision.DEFAULT,
    )
    logits_ref[...] = logits

    iota = jax.lax.broadcasted_iota(jnp.int32, (BT, E), 1)
    work = logits
    vals = []
    idxs = []
    for _ in range(K):
        m = jnp.max(work, axis=1, keepdims=True)
        hit = work >= m
        idx = jnp.min(jnp.where(hit, iota, E), axis=1, keepdims=True)
        vals.append(m)
        idxs.append(idx)
        work = jnp.where(iota == idx, -jnp.inf, work)
    v = jnp.concatenate(vals, axis=1)
    gates = jnp.exp(v - v[:, :1])
    gates = gates / jnp.sum(gates, axis=1, keepdims=True)
    idx_ref[...] = jnp.concatenate(idxs, axis=1)
    gates_ref[...] = gates


@functools.partial(jax.jit, static_argnames=())
def kernel(hidden_states, gate_weight):
    grid = (N_TOK // BT,)
    idx, gates, logits = pl.pallas_call(
        _router_block,
        grid=grid,
        in_specs=[
            pl.BlockSpec((BT, D), lambda i: (i, 0)),
            pl.BlockSpec((E, D), lambda i: (0, 0)),
        ],
        out_specs=[
            pl.BlockSpec((BT, K), lambda i: (i, 0)),
            pl.BlockSpec((BT, K), lambda i: (i, 0)),
            pl.BlockSpec((BT, E), lambda i: (i, 0)),
        ],
        out_shape=[
            jax.ShapeDtypeStruct((N_TOK, K), jnp.int32),
            jax.ShapeDtypeStruct((N_TOK, K), jnp.float32),
            jax.ShapeDtypeStruct((N_TOK, E), jnp.float32),
        ],
    )(hidden_states, gate_weight)
    return idx, gates, logits


# speedup vs baseline: 1.3709x; 1.0014x over previous
import functools

import jax
import jax.numpy as jnp
from jax.experimental import pallas as pl
from jax.experimental.pallas import tpu as pltpu

N_TOK = 16384
D = 2048
E = 64
K = 8
BT = 2048


def _router_block(x_ref, w_ref, idx_ref, gates_ref, logits_ref):
    x = x_ref[...]
    w = w_ref[...]
    logits = jax.lax.dot_general(
        x, w, (((1,), (1,)), ((), ())),
        preferred_element_type=jnp.float32,
        precision=jax.lax.Precision.DEFAULT,
    )
    logits_ref[...] = logits

    iota = jax.lax.broadcasted_iota(jnp.int32, (BT, E), 1)
    work = logits
    vals = []
    idxs = []
    for _ in range(K):
        m = jnp.max(work, axis=1, keepdims=True)
        hit = work >= m
        idx = jnp.min(jnp.where(hit, iota, E), axis=1, keepdims=True)
        vals.append(m)
        idxs.append(idx)
        work = jnp.where(iota == idx, -jnp.inf, work)
    v = jnp.concatenate(vals, axis=1)
    gates = jnp.exp(v - v[:, :1])
    gates = gates / jnp.sum(gates, axis=1, keepdims=True)
    idx_ref[...] = jnp.concatenate(idxs, axis=1)
    gates_ref[...] = gates


@functools.partial(jax.jit, static_argnames=())
def kernel(hidden_states, gate_weight):
    grid = (N_TOK // BT,)
    idx, gates, logits = pl.pallas_call(
        _router_block,
        grid=grid,
        in_specs=[
            pl.BlockSpec((BT, D), lambda i: (i, 0)),
            pl.BlockSpec((E, D), lambda i: (0, 0)),
        ],
        out_specs=[
            pl.BlockSpec((BT, K), lambda i: (i, 0)),
            pl.BlockSpec((BT, K), lambda i: (i, 0)),
            pl.BlockSpec((BT, E), lambda i: (i, 0)),
        ],
        out_shape=[
            jax.ShapeDtypeStruct((N_TOK, K), jnp.int32),
            jax.ShapeDtypeStruct((N_TOK, K), jnp.float32),
            jax.ShapeDtypeStruct((N_TOK, E), jnp.float32),
        ],
    )(hidden_states, gate_weight)
    return idx, gates, logits
